# baseline (speedup 1.0000x reference)
import functools

import jax
import jax.numpy as jnp
from jax import lax
from jax.experimental import pallas as pl
from jax.experimental.pallas import tpu as pltpu
from jax.experimental.pallas import tpu_sc as plsc

BATCH = 16384
DIM = 64
CHUNK = 128


def _gather_call(ids_grouped, embeddings, num_cores, b_per_w):
    n_chunks = b_per_w // CHUNK
    mesh = plsc.VectorSubcoreMesh(core_axis_name="c", subcore_axis_name="s")

    @functools.partial(
        pl.kernel,
        mesh=mesh,
        out_type=jax.ShapeDtypeStruct((BATCH, DIM), jnp.float32),
        compiler_params=pltpu.CompilerParams(use_tc_tiling_on_sc=False),
        scratch_types=[
            pltpu.VMEM((n_chunks, CHUNK), jnp.int32),
            pltpu.VMEM((b_per_w, DIM), jnp.float32),
            pltpu.SemaphoreType.DMA,
        ],
    )
    def k(ids_hbm, table_hbm, out_hbm, idx_v, rows_v, sem):
        wid = lax.axis_index("s") * num_cores + lax.axis_index("c")
        base = wid * b_per_w
        pltpu.sync_copy(ids_hbm.at[wid], idx_v)
        copies = []
        for j in range(n_chunks):
            copies.append(
                pltpu.async_copy(
                    table_hbm.at[idx_v.at[j]],
                    rows_v.at[pl.ds(j * CHUNK, CHUNK)],
                    sem,
                )
            )
        for c in copies:
            c.wait()
        pltpu.sync_copy(rows_v, out_hbm.at[pl.ds(base, b_per_w)])

    return k(ids_grouped, embeddings)


def kernel(style_ids, embeddings):
    info = plsc.get_sparse_core_info()
    n_workers = info.num_cores * info.num_subcores
    b_per_w = BATCH // n_workers
    ids_grouped = style_ids.astype(jnp.int32).reshape(
        n_workers, b_per_w // CHUNK, CHUNK
    )
    return _gather_call(ids_grouped, embeddings, info.num_cores, b_per_w)
